# baseline (speedup 1.0000x reference)
import jax
import jax.numpy as jnp
from jax import lax
from jax.experimental import pallas as pl
from jax.experimental.pallas import tpu as pltpu
from jax.experimental.pallas import tpu_sc as plsc

N_VOCAB = 100000
EMBED_DIM = 64
OFFSET = 20
BATCH = 4096

_info = plsc.get_sparse_core_info()
NC, NS, L = _info.num_cores, _info.num_subcores, _info.num_lanes
NW = NC * NS
BAGS_PER_W = BATCH // NW
IDX_PER_W = BAGS_PER_W * OFFSET
CHUNK_BAGS = 32
CHUNK_ROWS = CHUNK_BAGS * OFFSET
CHUNK_IDX_ROWS = CHUNK_ROWS // 128
N_CHUNKS = BAGS_PER_W // CHUNK_BAGS
VREGS_PER_ROW = EMBED_DIM // L


def _bag_kernel(idx_hbm, weight_hbm, out_hbm, idx_v, pair_v, rows_v, out_v, sem):
    wid = lax.axis_index("s") * NC + lax.axis_index("c")
    pltpu.sync_copy(
        idx_hbm.at[pl.ds(wid * IDX_PER_W, IDX_PER_W)],
        idx_v.at[pl.ds(0, IDX_PER_W)],
    )
    def shift_body(i, carry):
        sl = pl.ds(i * L, L)
        pair_v[sl] = lax.shift_right_logical(idx_v[sl], 1)
        return carry
    lax.fori_loop(0, IDX_PER_W // L, shift_body, 0)

    for c in range(N_CHUNKS):
        copies = [
            pltpu.async_copy(
                weight_hbm.at[pair_v.at[pl.ds((c * CHUNK_IDX_ROWS + j) * 128, 128)]],
                rows_v.at[pl.ds(j * 128, 128)],
                sem,
            )
            for j in range(CHUNK_IDX_ROWS)
        ]
        for cp in copies:
            cp.wait()

        def reduce_bag(b, carry):
            base = b * OFFSET
            ib = c * CHUNK_ROWS + base
            cols_a = (idx_v[pl.ds(ib, L)] & 1) * EMBED_DIM
            cols_b = (idx_v[pl.ds(ib + L, L)] & 1) * EMBED_DIM
            for v in range(VREGS_PER_ROW):
                acc = None
                for r in range(OFFSET):
                    cv = cols_a if r < L else cols_b
                    col = cv[r % L] + v * L
                    part = rows_v[base + r, pl.ds(col, L)]
                    acc = part if acc is None else acc + part
                out_v[b, pl.ds(v * L, L)] = acc
            return carry

        lax.fori_loop(0, CHUNK_BAGS, reduce_bag, 0)

        pltpu.sync_copy(
            out_v,
            out_hbm.at[pl.ds(wid * BAGS_PER_W + c * CHUNK_BAGS, CHUNK_BAGS)],
        )


@jax.jit
def _bag(idx, weight2):
    mesh = plsc.VectorSubcoreMesh(core_axis_name="c", subcore_axis_name="s")
    return pl.kernel(
        _bag_kernel,
        mesh=mesh,
        out_type=jax.ShapeDtypeStruct((BATCH, EMBED_DIM), jnp.float32),
        scratch_types=[
            pltpu.VMEM((IDX_PER_W + L,), jnp.int32),
            pltpu.VMEM((IDX_PER_W,), jnp.int32),
            pltpu.VMEM((CHUNK_ROWS, 2 * EMBED_DIM), jnp.float32),
            pltpu.VMEM((CHUNK_BAGS, EMBED_DIM), jnp.float32),
            pltpu.SemaphoreType.DMA,
        ],
    )(idx, weight2)


def kernel(sparse_index_group_batch, sparse_offset_group_batch, weight):
    del sparse_offset_group_batch
    idx = sparse_index_group_batch.astype(jnp.int32)
    weight2 = weight.reshape(N_VOCAB // 2, 2 * EMBED_DIM)
    return _bag(idx, weight2)


# speedup vs baseline: 1.7307x; 1.7307x over previous
import jax
import jax.numpy as jnp
from jax import lax
from jax.experimental import pallas as pl
from jax.experimental.pallas import tpu as pltpu
from jax.experimental.pallas import tpu_sc as plsc

N_VOCAB = 100000
EMBED_DIM = 64
OFFSET = 20
BATCH = 4096

_info = plsc.get_sparse_core_info()
NC, NS, L = _info.num_cores, _info.num_subcores, _info.num_lanes
NW = NC * NS
DIMS_PER_W = EMBED_DIM // NW
CHUNK_BAGS = 1024
CHUNK_IDX = CHUNK_BAGS * OFFSET
N_CHUNKS = BATCH // CHUNK_BAGS
GROUPS = CHUNK_BAGS // L


def _bag_kernel(idx_hbm, wt_hbm, out_hbm, slab_v, idx_v, acc_v):
    wid = lax.axis_index("s") * NC + lax.axis_index("c")
    lane_pos = lax.iota(jnp.int32, L) * OFFSET

    for d in range(DIMS_PER_W):
        c = wid * DIMS_PER_W + d
        pltpu.sync_copy(wt_hbm.at[c], slab_v)

        for ch in range(N_CHUNKS):
            pltpu.sync_copy(idx_hbm.at[pl.ds(ch * CHUNK_IDX, CHUNK_IDX)], idx_v)

            def group_body(g, carry):
                base = g * (L * OFFSET)
                acc = None
                for r in range(OFFSET):
                    pos = lane_pos + (base + r)
                    iv = plsc.load_gather(idx_v, [pos])
                    sv = plsc.load_gather(slab_v, [iv])
                    acc = sv if acc is None else acc + sv
                acc_v[pl.ds(ch * CHUNK_BAGS + g * L, L)] = acc
                return carry

            lax.fori_loop(0, GROUPS, group_body, 0)

        pltpu.sync_copy(acc_v, out_hbm.at[c])


@jax.jit
def _bag(idx, wt):
    mesh = plsc.VectorSubcoreMesh(core_axis_name="c", subcore_axis_name="s")
    return pl.kernel(
        _bag_kernel,
        mesh=mesh,
        compiler_params=pltpu.CompilerParams(needs_layout_passes=False),
        out_type=jax.ShapeDtypeStruct((EMBED_DIM, BATCH), jnp.float32),
        scratch_types=[
            pltpu.VMEM((N_VOCAB,), jnp.float32),
            pltpu.VMEM((CHUNK_IDX,), jnp.int32),
            pltpu.VMEM((BATCH,), jnp.float32),
        ],
    )(idx, wt)


def kernel(sparse_index_group_batch, sparse_offset_group_batch, weight):
    del sparse_offset_group_batch
    idx = sparse_index_group_batch.astype(jnp.int32)
    out_t = _bag(idx, weight.T)
    return out_t.T


# speedup vs baseline: 1.8677x; 1.0792x over previous
import jax
import jax.numpy as jnp
from jax import lax
from jax.experimental import pallas as pl
from jax.experimental.pallas import tpu as pltpu
from jax.experimental.pallas import tpu_sc as plsc

N_VOCAB = 100000
EMBED_DIM = 64
OFFSET = 20
BATCH = 4096

_info = plsc.get_sparse_core_info()
NC, NS, L = _info.num_cores, _info.num_subcores, _info.num_lanes
NW = NC * NS
DIMS_PER_W = EMBED_DIM // NW
CHUNK_BAGS = 512
CHUNK_IDX = CHUNK_BAGS * OFFSET
N_CHUNKS = BATCH // CHUNK_BAGS
GROUPS = CHUNK_BAGS // L


def _bag_kernel(idx_hbm, wt_hbm, out_hbm, slab_v, idxa_v, idxb_v, acc_v, sem):
    wid = lax.axis_index("s") * NC + lax.axis_index("c")
    lane_pos = lax.iota(jnp.int32, L) * OFFSET

    for d in range(DIMS_PER_W):
        c = wid * DIMS_PER_W + d
        pltpu.sync_copy(wt_hbm.at[c], slab_v)

        bufs = (idxa_v, idxb_v)
        copies = [
            pltpu.async_copy(
                idx_hbm.at[pl.ds(0 * CHUNK_IDX, CHUNK_IDX)], bufs[0], sem
            )
        ]
        for ch in range(N_CHUNKS):
            copies[ch].wait()
            if ch + 1 < N_CHUNKS:
                copies.append(
                    pltpu.async_copy(
                        idx_hbm.at[pl.ds((ch + 1) * CHUNK_IDX, CHUNK_IDX)],
                        bufs[(ch + 1) % 2],
                        sem,
                    )
                )
            idx_v = bufs[ch % 2]

            def group_body(g, carry):
                base = g * (L * OFFSET)
                acc = None
                for r in range(OFFSET):
                    pos = lane_pos + (base + r)
                    iv = plsc.load_gather(idx_v, [pos])
                    sv = plsc.load_gather(slab_v, [iv])
                    acc = sv if acc is None else acc + sv
                acc_v[pl.ds(ch * CHUNK_BAGS + g * L, L)] = acc
                return carry

            lax.fori_loop(0, GROUPS, group_body, 0)

        pltpu.sync_copy(acc_v, out_hbm.at[c])


@jax.jit
def _bag(idx, wt):
    mesh = plsc.VectorSubcoreMesh(core_axis_name="c", subcore_axis_name="s")
    return pl.kernel(
        _bag_kernel,
        mesh=mesh,
        compiler_params=pltpu.CompilerParams(needs_layout_passes=False),
        out_type=jax.ShapeDtypeStruct((EMBED_DIM, BATCH), jnp.float32),
        scratch_types=[
            pltpu.VMEM((N_VOCAB,), jnp.float32),
            pltpu.VMEM((CHUNK_IDX,), jnp.int32),
            pltpu.VMEM((CHUNK_IDX,), jnp.int32),
            pltpu.VMEM((BATCH,), jnp.float32),
            pltpu.SemaphoreType.DMA,
        ],
    )(idx, wt)


def kernel(sparse_index_group_batch, sparse_offset_group_batch, weight):
    del sparse_offset_group_batch
    idx = sparse_index_group_batch.astype(jnp.int32)
    out_t = _bag(idx, weight.T)
    return out_t.T


# speedup vs baseline: 2.0413x; 1.0929x over previous
import jax
import jax.numpy as jnp
from jax import lax
from jax.experimental import pallas as pl
from jax.experimental.pallas import tpu as pltpu
from jax.experimental.pallas import tpu_sc as plsc

N_VOCAB = 100000
EMBED_DIM = 64
OFFSET = 20
BATCH = 4096

_info = plsc.get_sparse_core_info()
NC, NS, L = _info.num_cores, _info.num_subcores, _info.num_lanes
NW = NC * NS
DIMS_PER_W = EMBED_DIM // NW
CHUNK_BAGS = 512
CHUNK_IDX = CHUNK_BAGS * OFFSET
N_CHUNKS = BATCH // CHUNK_BAGS
GROUPS = CHUNK_BAGS // L


def _bag_kernel(idx_hbm, wt_hbm, out_hbm, slab_v, idxa_v, idxb_v, acc_v, sem):
    wid = lax.axis_index("s") * NC + lax.axis_index("c")

    for d in range(DIMS_PER_W):
        c = wid * DIMS_PER_W + d
        pltpu.sync_copy(wt_hbm.at[c], slab_v)

        bufs = (idxa_v, idxb_v)
        copies = [
            pltpu.async_copy(
                idx_hbm.at[pl.ds(0 * CHUNK_IDX, CHUNK_IDX)], bufs[0], sem
            )
        ]
        for ch in range(N_CHUNKS):
            copies[ch].wait()
            if ch + 1 < N_CHUNKS:
                copies.append(
                    pltpu.async_copy(
                        idx_hbm.at[pl.ds((ch + 1) * CHUNK_IDX, CHUNK_IDX)],
                        bufs[(ch + 1) % 2],
                        sem,
                    )
                )
            idx_v = bufs[ch % 2]

            def group_body(g, carry):
                base = g * L
                acc = None
                for r in range(OFFSET):
                    iv = idx_v[pl.ds(r * CHUNK_BAGS + base, L)]
                    sv = plsc.load_gather(slab_v, [iv])
                    acc = sv if acc is None else acc + sv
                acc_v[pl.ds(ch * CHUNK_BAGS + g * L, L)] = acc
                return carry

            lax.fori_loop(0, GROUPS, group_body, 0)

        pltpu.sync_copy(acc_v, out_hbm.at[c])


@jax.jit
def _bag(idx, wt):
    mesh = plsc.VectorSubcoreMesh(core_axis_name="c", subcore_axis_name="s")
    return pl.kernel(
        _bag_kernel,
        mesh=mesh,
        compiler_params=pltpu.CompilerParams(needs_layout_passes=False),
        out_type=jax.ShapeDtypeStruct((EMBED_DIM, BATCH), jnp.float32),
        scratch_types=[
            pltpu.VMEM((N_VOCAB,), jnp.float32),
            pltpu.VMEM((CHUNK_IDX,), jnp.int32),
            pltpu.VMEM((CHUNK_IDX,), jnp.int32),
            pltpu.VMEM((BATCH,), jnp.float32),
            pltpu.SemaphoreType.DMA,
        ],
    )(idx, wt)


def kernel(sparse_index_group_batch, sparse_offset_group_batch, weight):
    del sparse_offset_group_batch
    idx = sparse_index_group_batch.astype(jnp.int32)
    idx_t = (
        idx.reshape(N_CHUNKS, CHUNK_BAGS, OFFSET)
        .transpose(0, 2, 1)
        .reshape(-1)
    )
    out_t = _bag(idx_t, weight.T)
    return out_t.T


# speedup vs baseline: 2.0777x; 1.0178x over previous
import jax
import jax.numpy as jnp
from jax import lax
from jax.experimental import pallas as pl
from jax.experimental.pallas import tpu as pltpu
from jax.experimental.pallas import tpu_sc as plsc

N_VOCAB = 100000
EMBED_DIM = 64
OFFSET = 20
BATCH = 4096

_info = plsc.get_sparse_core_info()
NC, NS, L = _info.num_cores, _info.num_subcores, _info.num_lanes
NW = NC * NS
DIMS_PER_W = EMBED_DIM // NW
CHUNK_BAGS = 512
CHUNK_IDX = CHUNK_BAGS * OFFSET
N_CHUNKS = BATCH // CHUNK_BAGS
GROUPS = CHUNK_BAGS // L


def _bag_kernel(idx_hbm, wt_hbm, out_hbm, slab_v, idxa_v, idxb_v, acc_v, sem):
    wid = lax.axis_index("s") * NC + lax.axis_index("c")

    for d in range(DIMS_PER_W):
        c = wid * DIMS_PER_W + d
        pltpu.sync_copy(wt_hbm.at[c], slab_v)

        bufs = (idxa_v, idxb_v)
        copies = [
            pltpu.async_copy(
                idx_hbm.at[pl.ds(0 * CHUNK_IDX, CHUNK_IDX)], bufs[0], sem
            )
        ]
        for ch in range(N_CHUNKS):
            copies[ch].wait()
            if ch + 1 < N_CHUNKS:
                copies.append(
                    pltpu.async_copy(
                        idx_hbm.at[pl.ds((ch + 1) * CHUNK_IDX, CHUNK_IDX)],
                        bufs[(ch + 1) % 2],
                        sem,
                    )
                )
            idx_v = bufs[ch % 2]

            @plsc.parallel_loop(0, GROUPS, step=1, unroll=2)
            def group_body(g):
                base = g * L
                parts = []
                for k in range(4):
                    acc = None
                    for r in range(5 * k, 5 * k + 5):
                        iv = idx_v[pl.ds(r * CHUNK_BAGS + base, L)]
                        sv = plsc.load_gather(slab_v, [iv])
                        acc = sv if acc is None else acc + sv
                    parts.append(acc)
                acc_v[pl.ds(ch * CHUNK_BAGS + base, L)] = (
                    (parts[0] + parts[1]) + (parts[2] + parts[3])
                )

        pltpu.sync_copy(acc_v, out_hbm.at[c])


@jax.jit
def _bag(idx, wt):
    mesh = plsc.VectorSubcoreMesh(core_axis_name="c", subcore_axis_name="s")
    return pl.kernel(
        _bag_kernel,
        mesh=mesh,
        compiler_params=pltpu.CompilerParams(needs_layout_passes=False),
        out_type=jax.ShapeDtypeStruct((EMBED_DIM, BATCH), jnp.float32),
        scratch_types=[
            pltpu.VMEM((N_VOCAB,), jnp.float32),
            pltpu.VMEM((CHUNK_IDX,), jnp.int32),
            pltpu.VMEM((CHUNK_IDX,), jnp.int32),
            pltpu.VMEM((BATCH,), jnp.float32),
            pltpu.SemaphoreType.DMA,
        ],
    )(idx, wt)


def kernel(sparse_index_group_batch, sparse_offset_group_batch, weight):
    del sparse_offset_group_batch
    idx = sparse_index_group_batch.astype(jnp.int32)
    idx_t = (
        idx.reshape(N_CHUNKS, CHUNK_BAGS, OFFSET)
        .transpose(0, 2, 1)
        .reshape(-1)
    )
    out_t = _bag(idx_t, weight.T)
    return out_t.T
